# baseline (speedup 1.0000x reference)
import functools

import jax
import jax.numpy as jnp
from jax import lax
from jax.experimental import pallas as pl
from jax.experimental.pallas import tpu as pltpu
from jax.experimental.pallas import tpu_sc as plsc

NC = 2
NS = 16
L = 16
NW = NC * NS

B, M, D = 4096, 200, 64
N = B * M
PER_W = N // NW
CR = 128
NCH = M
PITCH = D + 1
DBLK = D // 8
TILE = 8 * 128
OBUF = DBLK * TILE


def _sc_embed(x_perm, token_table, pos_table):
    mesh = plsc.VectorSubcoreMesh(core_axis_name="c", subcore_axis_name="s")

    @functools.partial(
        pl.kernel,
        out_type=jax.ShapeDtypeStruct((M, DBLK, NW, TILE), jnp.float32),
        mesh=mesh,
        scratch_types=[
            pltpu.VMEM((PER_W,), jnp.int32),
            [pltpu.VMEM((CR, D), jnp.float32) for _ in range(2)],
            pltpu.VMEM((CR * PITCH,), jnp.float32),
            [pltpu.VMEM((DBLK, TILE), jnp.float32) for _ in range(2)],
            pltpu.VMEM((M, D), jnp.float32),
            [pltpu.SemaphoreType.DMA for _ in range(2)],
            [pltpu.SemaphoreType.DMA for _ in range(2)],
        ],
        compiler_params=pltpu.CompilerParams(
            use_tc_tiling_on_sc=False, needs_layout_passes=False
        ),
    )
    def body(tok_hbm, idx_hbm, pos_hbm, out_hbm,
             idx_v, rows, skew, obuf, pos_v, gsems, osems):
        w = lax.axis_index("s") * NC + lax.axis_index("c")
        base = w * PER_W
        pltpu.sync_copy(pos_hbm, pos_v)
        pltpu.sync_copy(idx_hbm.at[pl.ds(base, PER_W)], idx_v)

        row_vecs = [lax.iota(jnp.int32, L) + bg * L for bg in range(8)]

        def fire_gather(g, buf, sem):
            off = (g % NCH) * CR
            return pltpu.async_copy(
                tok_hbm.at[idx_v.at[pl.ds(off, CR)]],
                rows[buf],
                sem,
            )

        def transpose_add(g, buf):
            m = g
            pvs = [pos_v[m, pl.ds(c16 * L, L)] for c16 in range(D // L)]

            @pl.loop(0, CR // 8)
            def _(r8):
                for rs in range(8):
                    r = r8 * 8 + rs
                    vs = [rows[buf][r, pl.ds(c16 * L, L)] + pvs[c16]
                          for c16 in range(D // L)]
                    for c16 in range(D // L):
                        skew[pl.ds(r * PITCH + c16 * L, L)] = vs[c16]

            @pl.loop(0, DBLK)
            def _(dblk):
                for dsub in range(8):
                    col = dblk * 8 + dsub
                    colv = jnp.broadcast_to(col, (L,))
                    o = dsub * 128
                    vs = [
                        plsc.load_gather(
                            skew, [row_vecs[bg] * PITCH + colv])
                        for bg in range(8)
                    ]
                    for bg in range(8):
                        obuf[buf][dblk, pl.ds(o + bg * L, L)] = vs[bg]

        def fire_outs(g, buf, sem):
            pltpu.async_copy(obuf[buf], out_hbm.at[g, :, w], sem)

        def drain_outs(buf, sem):
            pltpu.make_async_copy(obuf[buf], out_hbm.at[0, :, w], sem).wait()

        fire_gather(0, 0, gsems[0]).wait()

        @pl.loop(0, NCH // 2)
        def _(i):
            for b in range(2):
                g = 2 * i + b
                nb = 1 - b

                @pl.when(g >= 2)
                def _():
                    drain_outs(b, osems[b])

                nxt = fire_gather(g + 1, nb, gsems[nb])
                transpose_add(g, b)
                fire_outs(g, b, osems[b])
                nxt.wait()

        drain_outs(0, osems[0])
        drain_outs(1, osems[1])

    return body(token_table, x_perm, pos_table)


def kernel(x, token_table, pos_table):
    x_perm = (
        x.astype(jnp.int32)
        .reshape(NW, 128, NCH)
        .transpose(0, 2, 1)
        .reshape(-1)
    )
    out = _sc_embed(x_perm, token_table, pos_table)
    out5 = out.reshape(M, DBLK, NW, 8, 128)
    return out5.transpose(2, 4, 0, 1, 3).reshape(B, M, D)


# speedup vs baseline: 1.0056x; 1.0056x over previous
import functools

import jax
import jax.numpy as jnp
from jax import lax
from jax.experimental import pallas as pl
from jax.experimental.pallas import tpu as pltpu
from jax.experimental.pallas import tpu_sc as plsc

NC = 2
NS = 16
L = 16
NW = NC * NS

B, M, D = 4096, 200, 64
N = B * M
PER_W = N // NW
CR = 256
NCH = M // 2
PITCH = D + 1
DBLK = D // 8
TILE = 8 * 128
OBUF = DBLK * TILE


def _sc_embed(x_perm, token_table, pos_table):
    mesh = plsc.VectorSubcoreMesh(core_axis_name="c", subcore_axis_name="s")

    @functools.partial(
        pl.kernel,
        out_type=jax.ShapeDtypeStruct((NCH, 2 * DBLK, NW, TILE), jnp.float32),
        mesh=mesh,
        scratch_types=[
            pltpu.VMEM((PER_W,), jnp.int32),
            [pltpu.VMEM((CR, D), jnp.float32) for _ in range(2)],
            pltpu.VMEM((CR * PITCH,), jnp.float32),
            [pltpu.VMEM((2 * DBLK, TILE), jnp.float32) for _ in range(2)],
            pltpu.VMEM((M, D), jnp.float32),
            [pltpu.SemaphoreType.DMA for _ in range(2)],
            [pltpu.SemaphoreType.DMA for _ in range(2)],
        ],
        compiler_params=pltpu.CompilerParams(
            use_tc_tiling_on_sc=False, needs_layout_passes=False
        ),
    )
    def body(tok_hbm, idx_hbm, pos_hbm, out_hbm,
             idx_v, rows, skew, obuf, pos_v, gsems, osems):
        w = lax.axis_index("s") * NC + lax.axis_index("c")
        base = w * PER_W
        pltpu.sync_copy(pos_hbm, pos_v)
        pltpu.sync_copy(idx_hbm.at[pl.ds(base, PER_W)], idx_v)

        row_vecs = [lax.iota(jnp.int32, L) + bg * L for bg in range(8)]

        def fire_gather(g, buf, sem):
            off = (g % NCH) * CR
            return pltpu.async_copy(
                tok_hbm.at[idx_v.at[pl.ds(off, CR)]],
                rows[buf],
                sem,
            )

        def transpose_add(g, buf):
            for h in range(2):
                m = 2 * g + h
                pvs = [pos_v[m, pl.ds(c16 * L, L)] for c16 in range(D // L)]

                @pl.loop(0, 128 // 8)
                def _(r8):
                    for rs in range(8):
                        r = h * 128 + r8 * 8 + rs
                        vs = [rows[buf][r, pl.ds(c16 * L, L)] + pvs[c16]
                              for c16 in range(D // L)]
                        for c16 in range(D // L):
                            skew[pl.ds(r * PITCH + c16 * L, L)] = vs[c16]

            @pl.loop(0, DBLK)
            def _(dblk):
                for h in range(2):
                    for dsub in range(8):
                        col = dblk * 8 + dsub
                        colv = jnp.broadcast_to(col, (L,))
                        o = dsub * 128
                        hofs = h * 128 * PITCH
                        vs = [
                            plsc.load_gather(
                                skew,
                                [row_vecs[bg] * PITCH + (colv + hofs)])
                            for bg in range(8)
                        ]
                        for bg in range(8):
                            obuf[buf][h * DBLK + dblk,
                                      pl.ds(o + bg * L, L)] = vs[bg]

        def fire_outs(g, buf, sem):
            pltpu.async_copy(obuf[buf], out_hbm.at[g, :, w], sem)

        def drain_outs(buf, sem):
            pltpu.make_async_copy(obuf[buf], out_hbm.at[0, :, w], sem).wait()

        fire_gather(0, 0, gsems[0]).wait()

        @pl.loop(0, NCH // 2)
        def _(i):
            for b in range(2):
                g = 2 * i + b
                nb = 1 - b

                @pl.when(g >= 2)
                def _():
                    drain_outs(b, osems[b])

                nxt = fire_gather(g + 1, nb, gsems[nb])
                transpose_add(g, b)
                fire_outs(g, b, osems[b])
                nxt.wait()

        drain_outs(0, osems[0])
        drain_outs(1, osems[1])

    return body(token_table, x_perm, pos_table)


def kernel(x, token_table, pos_table):
    x_perm = (
        x.astype(jnp.int32)
        .reshape(NW, 128, M)
        .transpose(0, 2, 1)
        .reshape(-1)
    )
    out = _sc_embed(x_perm, token_table, pos_table)
    out5 = out.reshape(M, DBLK, NW, 8, 128)
    return out5.transpose(2, 4, 0, 1, 3).reshape(B, M, D)
